# baseline (speedup 1.0000x reference)
import dataclasses
import functools

import jax
import jax.numpy as jnp
from jax import lax
from jax.experimental import pallas as pl
from jax.experimental.pallas import tpu as pltpu
from jax.experimental.pallas import tpu_sc as plsc

_N_ENT = 10000
_N_USR = 10000
_D = 128
_NE = 320000
_NNZ = 160000
_NREL = 8
_NT = 16
_EB = 80
_RPT = 624
_TAIL0 = _RPT * _NT
_TAILN = _N_ENT - _TAIL0
_EDGES_PER_TILE = _NE // _NT
_NNZ_PER_TILE = _NNZ // _NT
_NB_E = _EDGES_PER_TILE // _EB
_NB_U = _NNZ_PER_TILE // _EB

_mesh = plsc.VectorSubcoreMesh(core_axis_name="core", subcore_axis_name="subcore")

_cp = pltpu.CompilerParams()
if "needs_layout_passes" in pltpu.CompilerParams.__dataclass_fields__:
  _cp = dataclasses.replace(_cp, needs_layout_passes=False)

_GDN = lax.GatherDimensionNumbers(
    offset_dims=(), collapsed_slice_dims=(0,), start_index_map=(0,))


def _vtake(vec, idx):
  return lax.gather(vec, idx[:, None], _GDN, slice_sizes=(1,),
                    mode=lax.GatherScatterMode.PROMISE_IN_BOUNDS)


def _make_sc(edges):
  nb = _NB_E if edges else _NB_U
  per_tile = _EDGES_PER_TILE if edges else _NNZ_PER_TILE

  scratch = [pltpu.VMEM_SHARED((_N_ENT, _D), jnp.float32)]
  if edges:
    scratch.append(pltpu.VMEM_SHARED((_N_ENT,), jnp.float32))
  scratch += (
      [pltpu.VMEM((_EB, _D), jnp.float32)] * 4
      + [pltpu.VMEM((_EB,), jnp.int32)] * 4
      + [pltpu.VMEM((_EB,), jnp.int32)] * 4
      + [pltpu.VMEM((_EB,), jnp.int32)] * 4
      + [pltpu.VMEM((_EB,), jnp.float32)] * 2
      + [pltpu.SemaphoreType.DMA] * 12
  )
  mesh1 = plsc.VectorSubcoreMesh(
      core_axis_name="core", subcore_axis_name="subcore", num_cores=1)

  def body(*refs):
    (src_hbm, gidx_hbm, sidx_hbm, aux_hbm, out_hbm) = refs[:5]
    rest = refs[5:]
    sums = rest[0]
    cnt = rest[1] if edges else None
    rest = rest[(2 if edges else 1):]
    ds_ = rest[0:4]
    ts_ = rest[4:8]
    hs_ = rest[8:12]
    as_ = rest[12:16]
    ones_v, cz_v = rest[16:18]
    isem = rest[18:22]
    gsem = rest[22:26]
    ssem = rest[26:30]
    tid = lax.axis_index("subcore")
    r0 = tid * _RPT
    d0 = ds_[0]

    @pl.loop(0, _EB, step=16)
    def _(i):
      cz_v[pl.ds(i, 16)] = jnp.zeros((16,), jnp.float32)
      ones_v[pl.ds(i, 16)] = jnp.ones((16,), jnp.float32)

    @pl.loop(0, _EB)
    def _(i):
      for k in range(_D // 16):
        d0[i, pl.ds(16 * k, 16)] = jnp.zeros((16,), jnp.float32)

    @pl.loop(0, 560, step=80)
    def _(c):
      pltpu.async_copy(d0, sums.at[pl.ds(r0 + c, _EB)], gsem[0])

    pltpu.async_copy(d0.at[pl.ds(0, 64)], sums.at[pl.ds(r0 + 560, 64)],
                     gsem[1])
    if edges:
      @pl.loop(0, 560, step=80)
      def _(c):
        pltpu.async_copy(cz_v, cnt.at[pl.ds(r0 + c, _EB)], gsem[2])

      pltpu.async_copy(cz_v.at[pl.ds(0, 64)], cnt.at[pl.ds(r0 + 560, 64)],
                       gsem[3])

    @pl.when(tid == _NT - 1)
    def _():
      pltpu.sync_copy(d0.at[pl.ds(0, _TAILN)], sums.at[pl.ds(_TAIL0, _TAILN)])
      if edges:
        pltpu.sync_copy(cz_v.at[pl.ds(0, _TAILN)],
                        cnt.at[pl.ds(_TAIL0, _TAILN)])

    @pl.loop(0, 560, step=80)
    def _(c):
      pltpu.make_async_copy(d0, sums.at[pl.ds(r0 + c, _EB)], gsem[0]).wait()

    pltpu.make_async_copy(d0.at[pl.ds(0, 64)], sums.at[pl.ds(r0 + 560, 64)],
                          gsem[1]).wait()
    if edges:
      @pl.loop(0, 560, step=80)
      def _(c):
        pltpu.make_async_copy(cz_v, cnt.at[pl.ds(r0 + c, _EB)],
                              gsem[2]).wait()

      pltpu.make_async_copy(cz_v.at[pl.ds(0, 64)],
                            cnt.at[pl.ds(r0 + 560, 64)], gsem[3]).wait()

    plsc.subcore_barrier()

    base = tid * per_tile

    def _xform(u):
      @pl.loop(0, _EB, step=16)
      def _(i):
        ts_[u][pl.ds(i, 16)] = (as_[u][pl.ds(i, 16)] * _N_ENT
                                + ts_[u][pl.ds(i, 16)])

    def _mul(dbuf, abuf):
      @pl.loop(0, _EB, step=16)
      def _(g):
        valv = plsc.bitcast(abuf[pl.ds(g, 16)], jnp.float32)
        for j in range(16):
          vb = _vtake(valv, jnp.full((16,), j, jnp.int32))
          e = g + j
          for k in range(_D // 16):
            dbuf[e, pl.ds(16 * k, 16)] = dbuf[e, pl.ds(16 * k, 16)] * vb

    def _issue_idx(b, u):
      off = pl.ds(base + b * _EB, _EB)
      pltpu.async_copy(gidx_hbm.at[off], ts_[u], isem[u])
      pltpu.async_copy(sidx_hbm.at[off], hs_[u], isem[u])
      pltpu.async_copy(aux_hbm.at[off], as_[u], isem[u])

    def _wait_idx(b, u):
      off = pl.ds(base + b * _EB, _EB)
      pltpu.make_async_copy(gidx_hbm.at[off], ts_[u], isem[u]).wait()
      pltpu.make_async_copy(sidx_hbm.at[off], hs_[u], isem[u]).wait()
      pltpu.make_async_copy(aux_hbm.at[off], as_[u], isem[u]).wait()

    def _issue_gather(u):
      if edges:
        _xform(u)
      pltpu.async_copy(src_hbm.at[ts_[u]], ds_[u], gsem[u])

    def _wait_gather(u):
      pltpu.make_async_copy(src_hbm.at[ts_[u]], ds_[u], gsem[u]).wait()

    def _issue_scat(u):
      pltpu.async_copy(ds_[u], sums.at[hs_[u]], ssem[u], add=True)
      if edges:
        pltpu.async_copy(ones_v, cnt.at[hs_[u]], ssem[u], add=True)

    def _wait_scat(u):
      pltpu.make_async_copy(ds_[u], sums.at[hs_[u]], ssem[u]).wait()
      if edges:
        pltpu.make_async_copy(ones_v, cnt.at[hs_[u]], ssem[u]).wait()

    def _maybe(cond, fn):
      if isinstance(cond, bool):
        if cond:
          fn()
      else:
        @pl.when(cond)
        def _():
          fn()

    def _section(b, u):
      _maybe(b + 1 < nb, lambda: _wait_idx(b + 1, (u + 1) % 4))
      _maybe(b >= 2, lambda: _wait_scat((u + 2) % 4))
      _maybe(b + 1 < nb, lambda: _issue_gather((u + 1) % 4))
      _maybe(b + 2 < nb, lambda: _issue_idx(b + 2, (u + 2) % 4))
      _wait_gather(u)
      if not edges:
        _mul(ds_[u], as_[u])
      _issue_scat(u)

    _issue_idx(0, 0)
    _issue_idx(1, 1)
    _wait_idx(0, 0)
    _issue_gather(0)

    nb4 = nb - (nb % 4)

    @pl.loop(0, nb4, step=4)
    def _(b0):
      for u in range(4):
        _section(b0 + u, u)

    for t in range(nb % 4):
      _section(nb4 + t, t)

    _wait_scat((nb - 2) % 4)
    _wait_scat((nb - 1) % 4)

    plsc.subcore_barrier()

    def _divide(dbuf, n):
      @pl.loop(0, n, step=16)
      def _(g):
        cv = cz_v[pl.ds(g, 16)]
        rv = 1.0 / jnp.maximum(cv, 1.0)
        for j in range(16):
          sb = _vtake(rv, jnp.full((16,), j, jnp.int32))
          e = g + j
          for k in range(_D // 16):
            dbuf[e, pl.ds(16 * k, 16)] = dbuf[e, pl.ds(16 * k, 16)] * sb

    def _chunk(c, n, u):
      pltpu.sync_copy(sums.at[pl.ds(r0 + c, n)], ds_[u].at[pl.ds(0, n)])
      if edges:
        pltpu.sync_copy(cnt.at[pl.ds(r0 + c, n)], cz_v.at[pl.ds(0, n)])
        _divide(ds_[u], n)
      pltpu.async_copy(ds_[u].at[pl.ds(0, n)], out_hbm.at[pl.ds(r0 + c, n)],
                       gsem[u])

    for ci in range(8):
      u = ci % 4
      n = _EB if ci < 7 else 64
      if ci >= 4:
        pltpu.make_async_copy(ds_[u].at[pl.ds(0, _EB)],
                              out_hbm.at[pl.ds(r0 + (ci - 4) * _EB, _EB)],
                              gsem[u]).wait()
      _chunk(ci * _EB, n, u)

    for ci in range(4, 8):
      u = ci % 4
      n = _EB if ci < 7 else 64
      pltpu.make_async_copy(ds_[u].at[pl.ds(0, n)],
                            out_hbm.at[pl.ds(r0 + ci * _EB, n)],
                            gsem[u]).wait()

    @pl.when(tid == _NT - 1)
    def _():
      pltpu.sync_copy(sums.at[pl.ds(_TAIL0, _TAILN)],
                      ds_[0].at[pl.ds(0, _TAILN)])
      if edges:
        pltpu.sync_copy(cnt.at[pl.ds(_TAIL0, _TAILN)],
                        cz_v.at[pl.ds(0, _TAILN)])
        _divide(ds_[0], _TAILN)
      pltpu.sync_copy(ds_[0].at[pl.ds(0, _TAILN)],
                      out_hbm.at[pl.ds(_TAIL0, _TAILN)])

  return pl.kernel(
      body,
      out_type=jax.ShapeDtypeStruct((_N_ENT, _D), jnp.float32),
      mesh=mesh1,
      compiler_params=_cp,
      scratch_types=scratch,
  )


_sc_edges = _make_sc(True)
_sc_users = _make_sc(False)


def _premul_body(ent_ref, w_ref, out_ref):
  r = pl.program_id(0)
  out_ref[...] = ent_ref[...] * w_ref[pl.ds(r, 1), :]


def _premul(entity_emb, weight):
  nb = _N_ENT // _BLK
  return pl.pallas_call(
      _premul_body,
      grid=(_NREL, nb),
      in_specs=[
          pl.BlockSpec((_BLK, _D), lambda r, i: (i, 0)),
          pl.BlockSpec((_NREL, _D), lambda r, i: (0, 0)),
      ],
      out_specs=pl.BlockSpec((_BLK, _D), lambda r, i: (r * nb + i, 0)),
      out_shape=jax.ShapeDtypeStruct((_NREL * _N_ENT, _D), jnp.float32),
  )(entity_emb, weight)


def _finish_body(usum_ref, user_ref, lat_ref, w_ref, datt_ref, uout_ref):
  logits = lax.dot_general(
      user_ref[...], lat_ref[...], (((1,), (1,)), ((), ())),
      precision=lax.Precision.HIGHEST, preferred_element_type=jnp.float32)
  score = jax.nn.softmax(logits, axis=1)
  dw = jax.nn.softmax(datt_ref[...], axis=-1)
  dw2 = lax.dot_general(
      dw, w_ref[...], (((1,), (0,)), ((), ())),
      precision=lax.Precision.HIGHEST, preferred_element_type=jnp.float32)
  mod = lax.dot_general(
      score, dw2, (((1,), (0,)), ((), ())),
      precision=lax.Precision.HIGHEST, preferred_element_type=jnp.float32)
  uout_ref[...] = usum_ref[...] * (1.0 + mod)


_BLK = 1000


def _finish(usum, user_emb, latent_emb, weight, disen_weight_att):
  n_blocks = _N_USR // _BLK
  return pl.pallas_call(
      _finish_body,
      grid=(n_blocks,),
      in_specs=[
          pl.BlockSpec((_BLK, _D), lambda i: (i, 0)),
          pl.BlockSpec((_BLK, _D), lambda i: (i, 0)),
          pl.BlockSpec((4, _D), lambda i: (0, 0)),
          pl.BlockSpec((_NREL, _D), lambda i: (0, 0)),
          pl.BlockSpec((4, _NREL), lambda i: (0, 0)),
      ],
      out_specs=pl.BlockSpec((_BLK, _D), lambda i: (i, 0)),
      out_shape=jax.ShapeDtypeStruct((_N_USR, _D), jnp.float32),
  )(usum, user_emb, latent_emb, weight, disen_weight_att)


def kernel(entity_emb, user_emb, latent_emb, edge_index, edge_type,
           interact_idx, interact_val, weight, disen_weight_att):
  head = edge_index[0].astype(jnp.int32)
  tail = edge_index[1].astype(jnp.int32)
  rel = ((edge_type.astype(jnp.int32) - 1) % _NREL).astype(jnp.int32)
  urow = interact_idx[0].astype(jnp.int32)
  ucol = interact_idx[1].astype(jnp.int32)
  uval_i = lax.bitcast_convert_type(interact_val, jnp.int32)
  tbl = _premul(entity_emb, weight)
  eagg = _sc_edges(tbl, tail, head, rel)
  usum = _sc_users(entity_emb, ucol, urow, uval_i)
  user_agg = _finish(usum, user_emb, latent_emb, weight, disen_weight_att)
  return (eagg, user_agg)


# speedup vs baseline: 1.0560x; 1.0560x over previous
import dataclasses
import functools

import jax
import jax.numpy as jnp
from jax import lax
from jax.experimental import pallas as pl
from jax.experimental.pallas import tpu as pltpu
from jax.experimental.pallas import tpu_sc as plsc

_N_ENT = 10000
_N_USR = 10000
_D = 128
_NE = 320000
_NNZ = 160000
_NREL = 8
_NT = 16
_EB = 80
_RPT = 624
_TAIL0 = _RPT * _NT
_TAILN = _N_ENT - _TAIL0
_EDGES_PER_TILE = _NE // _NT
_NNZ_PER_TILE = _NNZ // _NT
_NB_E = _EDGES_PER_TILE // _EB
_NB_U = _NNZ_PER_TILE // _EB

_mesh = plsc.VectorSubcoreMesh(core_axis_name="core", subcore_axis_name="subcore")

_cp = pltpu.CompilerParams()
if "needs_layout_passes" in pltpu.CompilerParams.__dataclass_fields__:
  _cp = dataclasses.replace(_cp, needs_layout_passes=False)

_GDN = lax.GatherDimensionNumbers(
    offset_dims=(), collapsed_slice_dims=(0,), start_index_map=(0,))


def _vtake(vec, idx):
  return lax.gather(vec, idx[:, None], _GDN, slice_sizes=(1,),
                    mode=lax.GatherScatterMode.PROMISE_IN_BOUNDS)


@functools.partial(
    pl.kernel,
    out_type=(
        jax.ShapeDtypeStruct((_N_ENT, _D), jnp.float32),
        jax.ShapeDtypeStruct((_N_USR, _D), jnp.float32),
    ),
    mesh=_mesh,
    compiler_params=_cp,
    scratch_types=[
        pltpu.VMEM_SHARED((_N_ENT, _D), jnp.float32),
        pltpu.VMEM_SHARED((_N_ENT,), jnp.float32),
        pltpu.VMEM((_EB, _D), jnp.float32),
        pltpu.VMEM((_EB, _D), jnp.float32),
        pltpu.VMEM((_EB, _D), jnp.float32),
        pltpu.VMEM((_EB, _D), jnp.float32),
        pltpu.VMEM((_EB,), jnp.int32),
        pltpu.VMEM((_EB,), jnp.int32),
        pltpu.VMEM((_EB,), jnp.int32),
        pltpu.VMEM((_EB,), jnp.int32),
        pltpu.VMEM((_EB,), jnp.int32),
        pltpu.VMEM((_EB,), jnp.int32),
        pltpu.VMEM((_EB,), jnp.int32),
        pltpu.VMEM((_EB,), jnp.int32),
        pltpu.VMEM((_EB,), jnp.int32),
        pltpu.VMEM((_EB,), jnp.int32),
        pltpu.VMEM((_EB,), jnp.int32),
        pltpu.VMEM((_EB,), jnp.int32),
        pltpu.VMEM((_EB,), jnp.float32),
        pltpu.VMEM((_EB,), jnp.float32),
    ] + [pltpu.SemaphoreType.DMA] * 12,
)
def _sc_agg(ent_hbm, tbl_hbm, head_hbm, tail_hbm, rel_hbm, urow_hbm,
            ucol_hbm, uval_hbm,
            eagg_hbm, usum_hbm,
            sums, cnt, d0, d1, d2, d3, t0, t1, t2, t3, h0, h1, h2, h3,
            a0, a1, a2, a3, ones_v, cz_v,
            si0, si1, si2, si3, sg0, sg1, sg2, sg3, ss0, ss1, ss2, ss3):
  tid = lax.axis_index("subcore")
  cid = lax.axis_index("core")
  r0 = tid * _RPT
  ds_ = (d0, d1, d2, d3)
  ts_ = (t0, t1, t2, t3)
  hs_ = (h0, h1, h2, h3)
  as_ = (a0, a1, a2, a3)
  isem = (si0, si1, si2, si3)
  gsem = (sg0, sg1, sg2, sg3)
  ssem = (ss0, ss1, ss2, ss3)

  @pl.loop(0, _EB, step=16)
  def _(i):
    cz_v[pl.ds(i, 16)] = jnp.zeros((16,), jnp.float32)
    ones_v[pl.ds(i, 16)] = jnp.ones((16,), jnp.float32)

  @pl.loop(0, _EB)
  def _(i):
    for k in range(_D // 16):
      d0[i, pl.ds(16 * k, 16)] = jnp.zeros((16,), jnp.float32)

  @pl.loop(0, 560, step=80)
  def _(c):
    pltpu.async_copy(d0, sums.at[pl.ds(r0 + c, _EB)], sg0)

  pltpu.async_copy(d0.at[pl.ds(0, 64)], sums.at[pl.ds(r0 + 560, 64)], sg1)

  @pl.when(cid == 0)
  def _zcnt():
    @pl.loop(0, 560, step=80)
    def _(c):
      pltpu.async_copy(cz_v, cnt.at[pl.ds(r0 + c, _EB)], sg2)

    pltpu.async_copy(cz_v.at[pl.ds(0, 64)], cnt.at[pl.ds(r0 + 560, 64)], sg3)

    @pl.when(tid == _NT - 1)
    def _():
      pltpu.sync_copy(cz_v.at[pl.ds(0, _TAILN)], cnt.at[pl.ds(_TAIL0, _TAILN)])

  @pl.when(tid == _NT - 1)
  def _():
    pltpu.sync_copy(d0.at[pl.ds(0, _TAILN)], sums.at[pl.ds(_TAIL0, _TAILN)])

  @pl.loop(0, 560, step=80)
  def _(c):
    pltpu.make_async_copy(d0, sums.at[pl.ds(r0 + c, _EB)], sg0).wait()

  pltpu.make_async_copy(d0.at[pl.ds(0, 64)], sums.at[pl.ds(r0 + 560, 64)],
                        sg1).wait()

  @pl.when(cid == 0)
  def _zcnt_wait():
    @pl.loop(0, 560, step=80)
    def _(c):
      pltpu.make_async_copy(cz_v, cnt.at[pl.ds(r0 + c, _EB)], sg2).wait()

    pltpu.make_async_copy(cz_v.at[pl.ds(0, 64)], cnt.at[pl.ds(r0 + 560, 64)],
                          sg3).wait()

  plsc.subcore_barrier()

  def _pump(nb, src_hbm, gidx_hbm, sidx_hbm, aux_hbm, base, mul_fn, xform_fn,
            with_cnt):
    def _issue_idx(b, u):
      off = pl.ds(base + b * _EB, _EB)
      pltpu.async_copy(gidx_hbm.at[off], ts_[u], isem[u])
      pltpu.async_copy(sidx_hbm.at[off], hs_[u], isem[u])
      if aux_hbm is not None:
        pltpu.async_copy(aux_hbm.at[off], as_[u], isem[u])

    def _wait_idx(b, u):
      off = pl.ds(base + b * _EB, _EB)
      pltpu.make_async_copy(gidx_hbm.at[off], ts_[u], isem[u]).wait()
      pltpu.make_async_copy(sidx_hbm.at[off], hs_[u], isem[u]).wait()
      if aux_hbm is not None:
        pltpu.make_async_copy(aux_hbm.at[off], as_[u], isem[u]).wait()

    def _issue_gather(u):
      if xform_fn is not None:
        xform_fn(u)
      pltpu.async_copy(src_hbm.at[ts_[u]], ds_[u], gsem[u])

    def _wait_gather(u):
      pltpu.make_async_copy(src_hbm.at[ts_[u]], ds_[u], gsem[u]).wait()

    def _issue_scat(u):
      pltpu.async_copy(ds_[u], sums.at[hs_[u]], ssem[u], add=True)
      if with_cnt:
        pltpu.async_copy(ones_v, cnt.at[hs_[u]], ssem[u], add=True)

    def _wait_scat(u):
      pltpu.make_async_copy(ds_[u], sums.at[hs_[u]], ssem[u]).wait()
      if with_cnt:
        pltpu.make_async_copy(ones_v, cnt.at[hs_[u]], ssem[u]).wait()

    def _maybe(cond, fn):
      if isinstance(cond, bool):
        if cond:
          fn()
      else:
        @pl.when(cond)
        def _():
          fn()

    def _section(b, u):
      _maybe(b + 1 < nb, lambda: _wait_idx(b + 1, (u + 1) % 4))
      _maybe(b >= 2, lambda: _wait_scat((u + 2) % 4))
      _maybe(b + 1 < nb, lambda: _issue_gather((u + 1) % 4))
      _maybe(b + 2 < nb, lambda: _issue_idx(b + 2, (u + 2) % 4))
      _wait_gather(u)
      if mul_fn is not None:
        mul_fn(ds_[u], as_[u], b)
      _issue_scat(u)

    _issue_idx(0, 0)
    _issue_idx(1, 1)
    _wait_idx(0, 0)
    _issue_gather(0)

    nb4 = nb - (nb % 4)

    @pl.loop(0, nb4, step=4)
    def _(b0):
      for u in range(4):
        _section(b0 + u, u)

    for t in range(nb % 4):
      _section(nb4 + t, t)

    _wait_scat((nb - 2) % 4)
    _wait_scat((nb - 1) % 4)

  def _xform_edges(u):
    @pl.loop(0, _EB, step=16)
    def _(i):
      ts_[u][pl.ds(i, 16)] = (as_[u][pl.ds(i, 16)] * _N_ENT
                              + ts_[u][pl.ds(i, 16)])

  def _mul_users(dbuf, abuf, b):
    @pl.loop(0, _EB, step=16)
    def _(g):
      valv = plsc.bitcast(abuf[pl.ds(g, 16)], jnp.float32)
      for j in range(16):
        vb = _vtake(valv, jnp.full((16,), j, jnp.int32))
        e = g + j
        for k in range(_D // 16):
          dbuf[e, pl.ds(16 * k, 16)] = dbuf[e, pl.ds(16 * k, 16)] * vb

  @pl.when(cid == 0)
  def _edges():
    _pump(_NB_E, tbl_hbm, tail_hbm, head_hbm, rel_hbm,
          tid * _EDGES_PER_TILE, None, _xform_edges, with_cnt=True)

  @pl.when(cid == 1)
  def _users():
    _pump(_NB_U, ent_hbm, ucol_hbm, urow_hbm, uval_hbm,
          tid * _NNZ_PER_TILE, _mul_users, None, with_cnt=False)

  plsc.subcore_barrier()

  def _divide(dbuf, n):
    @pl.loop(0, n, step=16)
    def _(g):
      cv = cz_v[pl.ds(g, 16)]
      rv = 1.0 / jnp.maximum(cv, 1.0)
      for j in range(16):
        sb = _vtake(rv, jnp.full((16,), j, jnp.int32))
        e = g + j
        for k in range(_D // 16):
          dbuf[e, pl.ds(16 * k, 16)] = dbuf[e, pl.ds(16 * k, 16)] * sb

  def _copy_out(dst_hbm, divide):
    def _chunk(c, n, u):
      pltpu.sync_copy(sums.at[pl.ds(r0 + c, n)], ds_[u].at[pl.ds(0, n)])
      if divide:
        pltpu.sync_copy(cnt.at[pl.ds(r0 + c, n)], cz_v.at[pl.ds(0, n)])
        _divide(ds_[u], n)
      pltpu.async_copy(ds_[u].at[pl.ds(0, n)], dst_hbm.at[pl.ds(r0 + c, n)],
                       gsem[u])

    for ci in range(8):
      u = ci % 4
      n = _EB if ci < 7 else 64
      if ci >= 4:
        pltpu.make_async_copy(ds_[u].at[pl.ds(0, _EB)],
                              dst_hbm.at[pl.ds(r0 + (ci - 4) * _EB, _EB)],
                              gsem[u]).wait()
      _chunk(ci * _EB, n, u)

    for ci in range(4, 8):
      u = ci % 4
      n = _EB if ci < 7 else 64
      pltpu.make_async_copy(ds_[u].at[pl.ds(0, n)],
                            dst_hbm.at[pl.ds(r0 + ci * _EB, n)],
                            gsem[u]).wait()

    @pl.when(tid == _NT - 1)
    def _():
      pltpu.sync_copy(sums.at[pl.ds(_TAIL0, _TAILN)],
                      ds_[0].at[pl.ds(0, _TAILN)])
      if divide:
        pltpu.sync_copy(cnt.at[pl.ds(_TAIL0, _TAILN)],
                        cz_v.at[pl.ds(0, _TAILN)])
        _divide(ds_[0], _TAILN)
      pltpu.sync_copy(ds_[0].at[pl.ds(0, _TAILN)],
                      dst_hbm.at[pl.ds(_TAIL0, _TAILN)])

  @pl.when(cid == 0)
  def _out_e():
    _copy_out(eagg_hbm, divide=True)

  @pl.when(cid == 1)
  def _out_u():
    _copy_out(usum_hbm, divide=False)


def _premul_body(ent_ref, w_ref, out_ref):
  r = pl.program_id(0)
  out_ref[...] = ent_ref[...] * w_ref[pl.ds(r, 1), :]


def _premul(entity_emb, weight):
  nb = _N_ENT // _BLK
  return pl.pallas_call(
      _premul_body,
      grid=(_NREL, nb),
      in_specs=[
          pl.BlockSpec((_BLK, _D), lambda r, i: (i, 0)),
          pl.BlockSpec((_NREL, _D), lambda r, i: (0, 0)),
      ],
      out_specs=pl.BlockSpec((_BLK, _D), lambda r, i: (r * nb + i, 0)),
      out_shape=jax.ShapeDtypeStruct((_NREL * _N_ENT, _D), jnp.float32),
  )(entity_emb, weight)


def _finish_body(usum_ref, user_ref, lat_ref, w_ref, datt_ref, uout_ref):
  logits = lax.dot_general(
      user_ref[...], lat_ref[...], (((1,), (1,)), ((), ())),
      precision=lax.Precision.HIGHEST, preferred_element_type=jnp.float32)
  score = jax.nn.softmax(logits, axis=1)
  dw = jax.nn.softmax(datt_ref[...], axis=-1)
  dw2 = lax.dot_general(
      dw, w_ref[...], (((1,), (0,)), ((), ())),
      precision=lax.Precision.HIGHEST, preferred_element_type=jnp.float32)
  mod = lax.dot_general(
      score, dw2, (((1,), (0,)), ((), ())),
      precision=lax.Precision.HIGHEST, preferred_element_type=jnp.float32)
  uout_ref[...] = usum_ref[...] * (1.0 + mod)


_BLK = 1000


def _finish(usum, user_emb, latent_emb, weight, disen_weight_att):
  n_blocks = _N_USR // _BLK
  return pl.pallas_call(
      _finish_body,
      grid=(n_blocks,),
      in_specs=[
          pl.BlockSpec((_BLK, _D), lambda i: (i, 0)),
          pl.BlockSpec((_BLK, _D), lambda i: (i, 0)),
          pl.BlockSpec((4, _D), lambda i: (0, 0)),
          pl.BlockSpec((_NREL, _D), lambda i: (0, 0)),
          pl.BlockSpec((4, _NREL), lambda i: (0, 0)),
      ],
      out_specs=pl.BlockSpec((_BLK, _D), lambda i: (i, 0)),
      out_shape=jax.ShapeDtypeStruct((_N_USR, _D), jnp.float32),
  )(usum, user_emb, latent_emb, weight, disen_weight_att)


def kernel(entity_emb, user_emb, latent_emb, edge_index, edge_type,
           interact_idx, interact_val, weight, disen_weight_att):
  head = edge_index[0].astype(jnp.int32)
  tail = edge_index[1].astype(jnp.int32)
  rel = ((edge_type.astype(jnp.int32) - 1) % _NREL).astype(jnp.int32)
  urow = interact_idx[0].astype(jnp.int32)
  ucol = interact_idx[1].astype(jnp.int32)
  uval_i = lax.bitcast_convert_type(interact_val, jnp.int32)
  tbl = _premul(entity_emb, weight)
  eagg, usum = _sc_agg(entity_emb, tbl, head, tail, rel, urow, ucol,
                       uval_i)
  user_agg = _finish(usum, user_emb, latent_emb, weight, disen_weight_att)
  return (eagg, user_agg)
